# Initial kernel scaffold; baseline (speedup 1.0000x reference)
import jax
import jax.numpy as jnp
from jax.experimental import pallas as pl


def kernel(z, edge_index):
    raise NotImplementedError("write your pallas kernel here")


# speedup vs baseline: 2.5264x; 2.5264x over previous
import functools

import jax
import jax.numpy as jnp
from jax import lax
from jax.experimental import pallas as pl
from jax.experimental.pallas import tpu as pltpu
from jax.experimental.pallas import tpu_sc as plsc

N = 10000
D = 128
E = 320000
L = 16
C = 128
NC, NS = 2, 16
NW = NC * NS
NCHUNKS = E // C
ITERS = (NCHUNKS + NW - 1) // NW


def _normalize(z):
    def body(z_ref, out_ref):
        zz = z_ref[...]
        norm = jnp.sqrt(jnp.sum(zz * zz, axis=1, keepdims=True))
        out_ref[...] = zz / jnp.maximum(norm, 1e-12)

    return pl.pallas_call(
        body,
        out_shape=jax.ShapeDtypeStruct((N, D), jnp.float32),
        grid=(10,),
        in_specs=[pl.BlockSpec((N // 10, D), lambda i: (i, 0))],
        out_specs=pl.BlockSpec((N // 10, D), lambda i: (i, 0)),
    )(z)


def _make_sc_kernel():
    mesh = plsc.VectorSubcoreMesh(core_axis_name="c", subcore_axis_name="s")

    @functools.partial(
        pl.kernel,
        out_type=jax.ShapeDtypeStruct((E,), jnp.float32),
        mesh=mesh,
        compiler_params=pltpu.CompilerParams(needs_layout_passes=False),
        scratch_types=[
            pltpu.VMEM((C,), jnp.int32),
            pltpu.VMEM((C,), jnp.int32),
            pltpu.VMEM((C, D), jnp.float32),
            pltpu.VMEM((C, D), jnp.float32),
            pltpu.VMEM((C,), jnp.float32),
            pltpu.SemaphoreType.DMA,
            pltpu.SemaphoreType.DMA,
        ],
    )
    def scorer(zn_hbm, ei_hbm, out_hbm, sidx, tidx, srows, trows, scores,
               sem_a, sem_b):
        w = lax.axis_index("s") * NC + lax.axis_index("c")
        lane = lax.iota(jnp.int32, L)

        def chunk(i, carry):
            c = w + i * NW

            @pl.when(c < NCHUNKS)
            def _():
                base = c * C
                pltpu.sync_copy(ei_hbm.at[0, pl.ds(base, C)], sidx)
                pltpu.sync_copy(ei_hbm.at[1, pl.ds(base, C)], tidx)
                cp_a = pltpu.async_copy(zn_hbm.at[sidx], srows, sem_a)
                cp_b = pltpu.async_copy(zn_hbm.at[tidx], trows, sem_b)
                cp_a.wait()
                cp_b.wait()
                for g in range(C // L):
                    dist = jnp.zeros((L,), jnp.float32)
                    for l in range(L):
                        e = g * L + l
                        acc = jnp.zeros((L,), jnp.float32)
                        for j in range(D // L):
                            a = srows[e, pl.ds(j * L, L)]
                            b = trows[e, pl.ds(j * L, L)]
                            d = a - b
                            acc = acc + d * d
                        dist = jnp.where(lane == l, jnp.sum(acc), dist)
                    scores[pl.ds(g * L, L)] = jnp.exp(-0.5 * dist)
                pltpu.sync_copy(scores, out_hbm.at[pl.ds(base, C)])

            return carry

        lax.fori_loop(0, ITERS, chunk, None)

    return scorer


_sc_score = _make_sc_kernel()


def kernel(z, edge_index):
    zn = _normalize(z.astype(jnp.float32))
    ei = edge_index.astype(jnp.int32)
    return _sc_score(zn, ei)


# speedup vs baseline: 4.0011x; 1.5837x over previous
import functools

import jax
import jax.numpy as jnp
from jax import lax
from jax.experimental import pallas as pl
from jax.experimental.pallas import tpu as pltpu
from jax.experimental.pallas import tpu_sc as plsc

N = 10000
D = 128
E = 320000
L = 16
C = 128
NC, NS = 2, 16
NW = NC * NS
EPW = E // NW
NFULL = EPW // C
TAIL = EPW - NFULL * C
TAIL_OFF = NFULL * C


def _normalize(z):
    def body(z_ref, out_ref):
        zz = z_ref[...]
        norm = jnp.sqrt(jnp.sum(zz * zz, axis=1, keepdims=True))
        out_ref[...] = zz / jnp.maximum(norm, 1e-12)

    return pl.pallas_call(
        body,
        out_shape=jax.ShapeDtypeStruct((N, D), jnp.float32),
        grid=(10,),
        in_specs=[pl.BlockSpec((N // 10, D), lambda i: (i, 0))],
        out_specs=pl.BlockSpec((N // 10, D), lambda i: (i, 0)),
    )(z)


def _make_sc_kernel():
    mesh = plsc.VectorSubcoreMesh(core_axis_name="c", subcore_axis_name="s")

    @functools.partial(
        pl.kernel,
        out_type=jax.ShapeDtypeStruct((E,), jnp.float32),
        mesh=mesh,
        compiler_params=pltpu.CompilerParams(needs_layout_passes=False),
        scratch_types=[
            pltpu.VMEM((EPW,), jnp.int32),
            pltpu.VMEM((EPW,), jnp.int32),
            pltpu.VMEM((C, D), jnp.float32),
            pltpu.VMEM((C, D), jnp.float32),
            pltpu.VMEM((C, D), jnp.float32),
            pltpu.VMEM((C, D), jnp.float32),
            pltpu.VMEM((EPW,), jnp.float32),
            pltpu.SemaphoreType.DMA,
            pltpu.SemaphoreType.DMA,
        ],
    )
    def scorer(zn_hbm, src_hbm, tgt_hbm, out_hbm, sidx, tidx, srows0, trows0,
               srows1, trows1, scores, sem0, sem1):
        w = lax.axis_index("s") * NC + lax.axis_index("c")
        base = w * EPW
        lane = lax.iota(jnp.int32, L)
        srows = (srows0, srows1)
        trows = (trows0, trows1)
        sems = (sem0, sem1)

        pltpu.sync_copy(src_hbm.at[pl.ds(base, EPW)], sidx)
        pltpu.sync_copy(tgt_hbm.at[pl.ds(base, EPW)], tidx)

        def start(i, b):
            off = pl.multiple_of(i * C, C)
            pltpu.async_copy(zn_hbm.at[sidx.at[pl.ds(off, C)]], srows[b],
                             sems[b])
            pltpu.async_copy(zn_hbm.at[tidx.at[pl.ds(off, C)]], trows[b],
                             sems[b])

        def wait(b):
            pltpu.make_async_copy(zn_hbm.at[pl.ds(0, C)], srows[b],
                                  sems[b]).wait()
            pltpu.make_async_copy(zn_hbm.at[pl.ds(0, C)], trows[b],
                                  sems[b]).wait()

        def dist_group(sref, tref, g):
            dist = jnp.zeros((L,), jnp.float32)
            for l in range(L):
                e = g * L + l
                acc = jnp.zeros((L,), jnp.float32)
                for j in range(D // L):
                    d = sref[e, pl.ds(j * L, L)] - tref[e, pl.ds(j * L, L)]
                    acc = acc + d * d
                dist = jnp.where(lane == l, jnp.sum(acc), dist)
            return dist

        def compute(i, b):
            coff = pl.multiple_of(i * C, C)
            for g in range(C // L):
                dist = dist_group(srows[b], trows[b], g)
                scores[pl.ds(coff + g * L, L)] = jnp.exp(-0.5 * dist)

        start(0, 0)
        start(1, 1)

        def pair(t, carry):
            for b in range(2):
                i = 2 * t + b
                wait(b)
                compute(i, b)

                @pl.when(i + 2 < NFULL)
                def _():
                    start(i + 2, b)

            return carry

        lax.fori_loop(0, NFULL // 2, pair, None)

        pltpu.sync_copy(zn_hbm.at[sidx.at[pl.ds(TAIL_OFF, TAIL)]],
                        srows0.at[pl.ds(0, TAIL)])
        pltpu.sync_copy(zn_hbm.at[tidx.at[pl.ds(TAIL_OFF, TAIL)]],
                        trows0.at[pl.ds(0, TAIL)])
        dist = dist_group(srows0, trows0, 0)
        scores[pl.ds(TAIL_OFF, TAIL)] = jnp.exp(-0.5 * dist)

        pltpu.sync_copy(scores, out_hbm.at[pl.ds(base, EPW)])

    return scorer


_sc_score = _make_sc_kernel()


def kernel(z, edge_index):
    zn = _normalize(z.astype(jnp.float32))
    ei = edge_index.astype(jnp.int32)
    return _sc_score(zn, ei[0], ei[1])
